# baseline (speedup 1.0000x reference)
import jax
import jax.numpy as jnp
import numpy as np
from jax import lax
from jax.experimental import pallas as pl
from jax.experimental.pallas import tpu as pltpu
from jax.experimental.pallas import tpu_sc as plsc

NC = 2
NS = 16
NW = NC * NS
C = 128
DEGW = 16


def _sc_mesh():
  return plsc.VectorSubcoreMesh(core_axis_name="c", subcore_axis_name="s",
                                num_cores=NC, num_subcores=NS)


def _make_deg_kernel(npad, k):
  rows_per_tile = npad // NS
  zc = rows_per_tile // C

  def body(dst_hbm, out_hbm, dst_v, ones_v, acc_sh, sem):
    c = lax.axis_index("c")
    s = lax.axis_index("s")
    wid = s * NC + c

    def fill_zero(i, _):
      ones_v[i, pl.ds(0, DEGW)] = jnp.zeros((DEGW,), jnp.float32)
      return 0
    lax.fori_loop(0, C, fill_zero, 0)
    for t in range(zc):
      pltpu.sync_copy(ones_v, acc_sh.at[pl.ds(s * rows_per_tile + t * C, C)])

    def fill_one(i, _):
      ones_v[i, pl.ds(0, DEGW)] = jnp.ones((DEGW,), jnp.float32)
      return 0
    lax.fori_loop(0, C, fill_one, 0)

    pltpu.sync_copy(dst_hbm.at[wid], dst_v)
    plsc.subcore_barrier()

    def step(j, _):
      pltpu.async_copy(ones_v, acc_sh.at[dst_v.at[j]], sem, add=True)
      return 0
    lax.fori_loop(0, k, step, 0)

    def drain(j, _):
      pltpu.make_async_copy(ones_v, acc_sh.at[dst_v.at[0]], sem).wait()
      return 0
    lax.fori_loop(0, k, drain, 0)

    plsc.subcore_barrier()
    pltpu.sync_copy(acc_sh.at[pl.ds(s * rows_per_tile, rows_per_tile)],
                    out_hbm.at[c, pl.ds(s * rows_per_tile, rows_per_tile)])

  return pl.kernel(
      body,
      out_type=jax.ShapeDtypeStruct((NC, npad, DEGW), jnp.float32),
      mesh=_sc_mesh(),
      scratch_types=[
          pltpu.VMEM((k, C), jnp.int32),
          pltpu.VMEM((C, DEGW), jnp.float32),
          pltpu.VMEM_SHARED((npad, DEGW), jnp.float32),
          pltpu.SemaphoreType.DMA,
      ],
      compiler_params=pltpu.CompilerParams(use_tc_tiling_on_sc=False),
      name="sc_degree",
  )


NBUF = 8
SLAG = 4


def _make_mp_kernel(npad, k, hh):
  assert k % NBUF == 0 and k >= 2 * NBUF
  rows_per_tile = npad // NS
  zc = rows_per_tile // C

  def body(table_hbm, src_hbm, dst_hbm, out_hbm, src_v, dst_v, *rest):
    bufs = rest[:NBUF]
    acc_sh = rest[NBUF]
    table_sh = rest[NBUF + 1]
    gsem = rest[NBUF + 2:NBUF + 2 + NBUF]
    ssem = rest[NBUF + 2 + NBUF:]
    c = lax.axis_index("c")
    s = lax.axis_index("s")
    wid = s * NC + c

    def gather(j, b):
      pltpu.async_copy(table_sh.at[src_v.at[j]], bufs[b], gsem[b])

    def wait_gather(b):
      pltpu.make_async_copy(table_sh.at[src_v.at[0]], bufs[b], gsem[b]).wait()

    def scatter(j, b):
      pltpu.async_copy(bufs[b], acc_sh.at[dst_v.at[j]], ssem[b], add=True)

    def wait_scatter(b):
      pltpu.make_async_copy(bufs[b], acc_sh.at[dst_v.at[0]], ssem[b]).wait()

    def fill_zero(i, _):
      for t in range(hh // 16):
        bufs[0][i, pl.ds(16 * t, 16)] = jnp.zeros((16,), jnp.float32)
      return 0
    lax.fori_loop(0, C, fill_zero, 0)
    for t in range(zc):
      pltpu.sync_copy(bufs[0], acc_sh.at[pl.ds(s * rows_per_tile + t * C, C)])

    pltpu.sync_copy(table_hbm.at[pl.ds(s * rows_per_tile, rows_per_tile)],
                    table_sh.at[pl.ds(s * rows_per_tile, rows_per_tile)])
    pltpu.sync_copy(src_hbm.at[wid], src_v)
    pltpu.sync_copy(dst_hbm.at[wid], dst_v)
    plsc.subcore_barrier()

    for b in range(NBUF):
      gather(b, b)
    for j in range(SLAG):
      wait_gather(j)
      scatter(j, j)

    def step(i, _):
      j0 = SLAG + NBUF * i
      for t in range(NBUF):
        b = (SLAG + t) % NBUF
        wait_gather(b)
        scatter(j0 + t, b)
        bn = (b + SLAG) % NBUF
        wait_scatter(bn)
        gather(j0 + t + SLAG, bn)
      return 0
    lax.fori_loop(0, (k - 2 * SLAG) // NBUF, step, 0)

    for j in range(k - SLAG, k):
      b = j % NBUF
      wait_gather(b)
      scatter(j, b)
      wait_scatter((b + SLAG) % NBUF)
    for j in range(k - SLAG, k):
      wait_scatter(j % NBUF)

    plsc.subcore_barrier()
    pltpu.sync_copy(acc_sh.at[pl.ds(s * rows_per_tile, rows_per_tile)],
                    out_hbm.at[c, pl.ds(s * rows_per_tile, rows_per_tile)])

  return pl.kernel(
      body,
      out_type=jax.ShapeDtypeStruct((NC, npad, hh), jnp.float32),
      mesh=_sc_mesh(),
      scratch_types=(
          [pltpu.VMEM((k, C), jnp.int32),
           pltpu.VMEM((k, C), jnp.int32)]
          + [pltpu.VMEM((C, hh), jnp.float32) for _ in range(NBUF)]
          + [pltpu.VMEM_SHARED((npad, hh), jnp.float32),
             pltpu.VMEM_SHARED((npad, hh), jnp.float32)]
          + [pltpu.SemaphoreType.DMA for _ in range(2 * NBUF)]
      ),
      compiler_params=pltpu.CompilerParams(use_tc_tiling_on_sc=False),
      name="sc_msgpass",
  )


PREC = lax.Precision.HIGHEST


def _blockdiag4(w):
  m, j = w.shape
  tiled = jnp.concatenate([jnp.concatenate([w] * 4, axis=1)] * 4, axis=0)
  rowq = lax.broadcasted_iota(jnp.int32, (4 * m, 4 * j), 0) // m
  colq = lax.broadcasted_iota(jnp.int32, (4 * m, 4 * j), 1) // j
  return jnp.where(rowq == colq, tiled, 0.0)


def _tile4(row):
  return jnp.concatenate([row] * 4, axis=1)


def _dinv_packed(deg_ref, n_real, n4):
  deg = 1.0 + deg_ref[0] + deg_ref[1]
  riota = lax.broadcasted_iota(jnp.int32, (n4, 1), 0)
  cols = []
  for q in range(4):
    dq = deg[:, DEGW * q:DEGW * q + 1]
    valid = riota * 4 + q < n_real
    dq = jnp.where(valid, lax.rsqrt(dq), 0.0)
    cols.append(jnp.broadcast_to(dq, (n4, 32)))
  return jnp.concatenate(cols, axis=1)


def _tc_prologue(n_real, npad):
  n4 = npad // 4

  def body(x4_ref, w1_ref, b1_ref, deg_ref,
           hplo_ref, hphi_ref, stlo_ref, sthi_ref, dinv_ref):
    x4 = x4_ref[...]
    w1 = w1_ref[...]
    hh = w1.shape[1] // 2
    dinv_p = _dinv_packed(deg_ref, n_real, n4)
    deginv_p = dinv_p * dinv_p
    h1lo = jnp.dot(x4, _blockdiag4(w1[:, :hh]),
                   preferred_element_type=jnp.float32, precision=PREC)
    h1hi = jnp.dot(x4, _blockdiag4(w1[:, hh:]),
                   preferred_element_type=jnp.float32, precision=PREC)
    b1 = b1_ref[...]
    hplo_ref[...] = dinv_p * h1lo
    hphi_ref[...] = dinv_p * h1hi
    stlo_ref[...] = deginv_p * h1lo + _tile4(b1[:, :hh])
    sthi_ref[...] = deginv_p * h1hi + _tile4(b1[:, hh:])
    dinv_ref[...] = dinv_p

  def call(x4, w1, b1_2d, deg4):
    return pl.pallas_call(
        body,
        out_shape=[jax.ShapeDtypeStruct((n4, 128), jnp.float32)
                   for _ in range(5)],
        name="tc_prologue",
        compiler_params=pltpu.CompilerParams(vmem_limit_bytes=100 * 1024 * 1024),
    )(x4, w1, b1_2d, deg4)

  return call


def _tc_mid(npad):
  n4 = npad // 4

  def body(acclo_ref, acchi_ref, stlo_ref, sthi_ref, dinv_ref, w2_ref,
           b2_ref, hplo_ref, hphi_ref, st2lo_ref, st2hi_ref):
    acclo = acclo_ref[...]
    acchi = acchi_ref[...]
    dinv_p = dinv_ref[...]
    deginv_p = dinv_p * dinv_p
    w2 = w2_ref[...]
    hh = w2.shape[1] // 2
    out1lo = jnp.maximum(dinv_p * (acclo[0] + acclo[1]) + stlo_ref[...], 0.0)
    out1hi = jnp.maximum(dinv_p * (acchi[0] + acchi[1]) + sthi_ref[...], 0.0)
    h2lo = (jnp.dot(out1lo, _blockdiag4(w2[:hh, :hh]),
                    preferred_element_type=jnp.float32, precision=PREC)
            + jnp.dot(out1hi, _blockdiag4(w2[hh:, :hh]),
                      preferred_element_type=jnp.float32, precision=PREC))
    h2hi = (jnp.dot(out1lo, _blockdiag4(w2[:hh, hh:]),
                    preferred_element_type=jnp.float32, precision=PREC)
            + jnp.dot(out1hi, _blockdiag4(w2[hh:, hh:]),
                      preferred_element_type=jnp.float32, precision=PREC))
    b2 = b2_ref[...]
    hplo_ref[...] = dinv_p * h2lo
    hphi_ref[...] = dinv_p * h2hi
    st2lo_ref[...] = deginv_p * h2lo + _tile4(b2[:, :hh])
    st2hi_ref[...] = deginv_p * h2hi + _tile4(b2[:, hh:])

  def call(acc_lo4, acc_hi4, stlo, sthi, dinvp, w2, b2_2d):
    return pl.pallas_call(
        body,
        out_shape=[jax.ShapeDtypeStruct((n4, 128), jnp.float32)
                   for _ in range(4)],
        name="tc_mid",
        compiler_params=pltpu.CompilerParams(vmem_limit_bytes=100 * 1024 * 1024),
    )(acc_lo4, acc_hi4, stlo, sthi, dinvp, w2, b2_2d)

  return call


def _tc_head(npad, g):
  n4 = npad // 4

  def body(acclo_ref, acchi_ref, st2lo_ref, st2hi_ref, dinv_ref, batchq_ref,
           wfc_ref, bfc_ref, y_ref):
    acclo = acclo_ref[...]
    acchi = acchi_ref[...]
    dinv_p = dinv_ref[...]
    out2lo = jnp.maximum(dinv_p * (acclo[0] + acclo[1]) + st2lo_ref[...], 0.0)
    out2hi = jnp.maximum(dinv_p * (acchi[0] + acchi[1]) + st2hi_ref[...], 0.0)
    gids = lax.broadcasted_iota(jnp.int32, (g, n4), 0)
    pooled_lo = jnp.zeros((g, 32), jnp.float32)
    pooled_hi = jnp.zeros((g, 32), jnp.float32)
    cnt = jnp.zeros((g, 1), jnp.float32)
    for q in range(4):
      oh = jnp.where(gids == batchq_ref[q:q + 1, :], 1.0, 0.0)
      mlo = jnp.dot(oh, out2lo, preferred_element_type=jnp.float32,
                    precision=PREC)
      mhi = jnp.dot(oh, out2hi, preferred_element_type=jnp.float32,
                    precision=PREC)
      pooled_lo = pooled_lo + mlo[:, 32 * q:32 * q + 32]
      pooled_hi = pooled_hi + mhi[:, 32 * q:32 * q + 32]
      cnt = cnt + jnp.sum(oh, axis=1, keepdims=True)
    pooled = jnp.concatenate([pooled_lo, pooled_hi], axis=1)
    pooled = pooled / jnp.maximum(cnt, 1.0)
    y_ref[...] = (
        jnp.dot(pooled, wfc_ref[...], preferred_element_type=jnp.float32,
                precision=PREC)
        + bfc_ref[...])

  def call(acc_lo4, acc_hi4, st2lo, st2hi, dinvp, batchq, wfc, bfc_2d):
    out = wfc.shape[1]
    return pl.pallas_call(
        body,
        out_shape=jax.ShapeDtypeStruct((g, out), jnp.float32),
        name="tc_head",
        compiler_params=pltpu.CompilerParams(vmem_limit_bytes=100 * 1024 * 1024),
    )(acc_lo4, acc_hi4, st2lo, st2hi, dinvp, batchq, wfc, bfc_2d)

  return call


def kernel(x, edge_index, batch, W1, b1, W2, b2, Wfc, bfc):
  n, din = x.shape
  e = edge_index.shape[1]
  h = W1.shape[1]
  g = 128

  npad = int(np.ceil(n / (NS * C))) * NS * C
  k = NBUF * int(np.ceil(e / (NW * C * NBUF)))
  cap = NW * k * C
  padrow = npad - 1

  src = edge_index[0]
  dst = edge_index[1]
  pad_idx = jnp.full((cap - e,), padrow, jnp.int32)
  src_r = jnp.concatenate([src, pad_idx]).reshape(NW, k, C)
  dst_r = jnp.concatenate([dst, pad_idx]).reshape(NW, k, C)

  n4 = npad // 4
  hh = h // 2
  x4 = jnp.zeros((npad, din), x.dtype).at[:n].set(x).reshape(n4, 4 * din)
  batch_pad = jnp.full((npad,), g, jnp.int32).at[:n].set(batch)
  batchq = batch_pad.reshape(n4, 4).T
  b1_2d = b1.reshape(1, -1)
  b2_2d = b2.reshape(1, -1)
  bfc_2d = bfc.reshape(1, -1)

  deg_fn = _make_deg_kernel(npad, k)
  mp_fn = _make_mp_kernel(npad, k, hh)

  def unpack(t):
    return t.reshape(npad, hh)

  def pack(a):
    return a.reshape(NC, n4, 128)

  degtab = deg_fn(dst_r)
  deg4 = degtab.reshape(NC, n4, 4 * DEGW)
  hp1_lo, hp1_hi, st1_lo, st1_hi, dinvp = _tc_prologue(n, npad)(
      x4, W1, b1_2d, deg4)
  acc1_lo = pack(mp_fn(unpack(hp1_lo), src_r, dst_r))
  acc1_hi = pack(mp_fn(unpack(hp1_hi), src_r, dst_r))
  hp2_lo, hp2_hi, st2_lo, st2_hi = _tc_mid(npad)(
      acc1_lo, acc1_hi, st1_lo, st1_hi, dinvp, W2, b2_2d)
  acc2_lo = pack(mp_fn(unpack(hp2_lo), src_r, dst_r))
  acc2_hi = pack(mp_fn(unpack(hp2_hi), src_r, dst_r))
  return _tc_head(npad, g)(acc2_lo, acc2_hi, st2_lo, st2_hi, dinvp, batchq,
                           Wfc, bfc_2d)


# speedup vs baseline: 1.0414x; 1.0414x over previous
import jax
import jax.numpy as jnp
import numpy as np
from jax import lax
from jax.experimental import pallas as pl
from jax.experimental.pallas import tpu as pltpu
from jax.experimental.pallas import tpu_sc as plsc

NC = 2
NS = 16
NW = NC * NS
C = 128
DEGW = 16


def _sc_mesh():
  return plsc.VectorSubcoreMesh(core_axis_name="c", subcore_axis_name="s",
                                num_cores=NC, num_subcores=NS)


def _make_deg_kernel(npad, k):
  rows_per_tile = npad // NS
  zc = rows_per_tile // C

  def body(ei_hbm, out_hbm, dst_v, ones_v, acc_sh, sem):
    c = lax.axis_index("c")
    s = lax.axis_index("s")
    wid = s * NC + c

    def fill_zero(i, _):
      ones_v[i, pl.ds(0, DEGW)] = jnp.zeros((DEGW,), jnp.float32)
      return 0
    lax.fori_loop(0, C, fill_zero, 0)
    for t in range(zc):
      pltpu.sync_copy(ones_v, acc_sh.at[pl.ds(s * rows_per_tile + t * C, C)])

    def fill_one(i, _):
      ones_v[i, pl.ds(0, DEGW)] = jnp.ones((DEGW,), jnp.float32)
      return 0
    lax.fori_loop(0, C, fill_one, 0)

    pltpu.sync_copy(ei_hbm.at[1, wid], dst_v)
    plsc.subcore_barrier()

    def step(j, _):
      pltpu.async_copy(ones_v, acc_sh.at[dst_v.at[j]], sem, add=True)
      return 0
    lax.fori_loop(0, k, step, 0)

    def drain(j, _):
      pltpu.make_async_copy(ones_v, acc_sh.at[dst_v.at[0]], sem).wait()
      return 0
    lax.fori_loop(0, k, drain, 0)

    plsc.subcore_barrier()
    pltpu.sync_copy(acc_sh.at[pl.ds(s * rows_per_tile, rows_per_tile)],
                    out_hbm.at[c, pl.ds(s * rows_per_tile, rows_per_tile)])

  return pl.kernel(
      body,
      out_type=jax.ShapeDtypeStruct((NC, npad, DEGW), jnp.float32),
      mesh=_sc_mesh(),
      scratch_types=[
          pltpu.VMEM((k, C), jnp.int32),
          pltpu.VMEM((C, DEGW), jnp.float32),
          pltpu.VMEM_SHARED((npad, DEGW), jnp.float32),
          pltpu.SemaphoreType.DMA,
      ],
      compiler_params=pltpu.CompilerParams(use_tc_tiling_on_sc=False),
      name="sc_degree",
  )


NBUF = 8
SLAG = 4


def _make_mp_kernel(npad, k, hh):
  assert k % NBUF == 0 and k >= 2 * NBUF
  rows_per_tile = npad // NS
  zc = rows_per_tile // C

  def body(table_hbm, ei_hbm, out_hbm, src_v, dst_v, *rest):
    bufs = rest[:NBUF]
    acc_sh = rest[NBUF]
    table_sh = rest[NBUF + 1]
    gsem = rest[NBUF + 2:NBUF + 2 + NBUF]
    ssem = rest[NBUF + 2 + NBUF:]
    c = lax.axis_index("c")
    s = lax.axis_index("s")
    wid = s * NC + c

    def gather(j, b):
      pltpu.async_copy(table_sh.at[src_v.at[j]], bufs[b], gsem[b])

    def wait_gather(b):
      pltpu.make_async_copy(table_sh.at[src_v.at[0]], bufs[b], gsem[b]).wait()

    def scatter(j, b):
      pltpu.async_copy(bufs[b], acc_sh.at[dst_v.at[j]], ssem[b], add=True)

    def wait_scatter(b):
      pltpu.make_async_copy(bufs[b], acc_sh.at[dst_v.at[0]], ssem[b]).wait()

    def fill_zero(i, _):
      for t in range(hh // 16):
        bufs[0][i, pl.ds(16 * t, 16)] = jnp.zeros((16,), jnp.float32)
      return 0
    lax.fori_loop(0, C, fill_zero, 0)
    for t in range(zc):
      pltpu.sync_copy(bufs[0], acc_sh.at[pl.ds(s * rows_per_tile + t * C, C)])

    pltpu.sync_copy(table_hbm.at[pl.ds(s * rows_per_tile, rows_per_tile)],
                    table_sh.at[pl.ds(s * rows_per_tile, rows_per_tile)])
    pltpu.sync_copy(ei_hbm.at[0, wid], src_v)
    pltpu.sync_copy(ei_hbm.at[1, wid], dst_v)
    plsc.subcore_barrier()

    for b in range(NBUF):
      gather(b, b)
    for j in range(SLAG):
      wait_gather(j)
      scatter(j, j)

    def step(i, _):
      j0 = SLAG + NBUF * i
      for t in range(NBUF):
        b = (SLAG + t) % NBUF
        wait_gather(b)
        scatter(j0 + t, b)
        bn = (b + SLAG) % NBUF
        wait_scatter(bn)
        gather(j0 + t + SLAG, bn)
      return 0
    lax.fori_loop(0, (k - 2 * SLAG) // NBUF, step, 0)

    for j in range(k - SLAG, k):
      b = j % NBUF
      wait_gather(b)
      scatter(j, b)
      wait_scatter((b + SLAG) % NBUF)
    for j in range(k - SLAG, k):
      wait_scatter(j % NBUF)

    plsc.subcore_barrier()
    pltpu.sync_copy(acc_sh.at[pl.ds(s * rows_per_tile, rows_per_tile)],
                    out_hbm.at[c, pl.ds(s * rows_per_tile, rows_per_tile)])

  return pl.kernel(
      body,
      out_type=jax.ShapeDtypeStruct((NC, npad, hh), jnp.float32),
      mesh=_sc_mesh(),
      scratch_types=(
          [pltpu.VMEM((k, C), jnp.int32),
           pltpu.VMEM((k, C), jnp.int32)]
          + [pltpu.VMEM((C, hh), jnp.float32) for _ in range(NBUF)]
          + [pltpu.VMEM_SHARED((npad, hh), jnp.float32),
             pltpu.VMEM_SHARED((npad, hh), jnp.float32)]
          + [pltpu.SemaphoreType.DMA for _ in range(2 * NBUF)]
      ),
      compiler_params=pltpu.CompilerParams(use_tc_tiling_on_sc=False),
      name="sc_msgpass",
  )


PREC = lax.Precision.HIGHEST


def _blockdiag4(w):
  m, j = w.shape
  tiled = jnp.concatenate([jnp.concatenate([w] * 4, axis=1)] * 4, axis=0)
  rowq = lax.broadcasted_iota(jnp.int32, (4 * m, 4 * j), 0) // m
  colq = lax.broadcasted_iota(jnp.int32, (4 * m, 4 * j), 1) // j
  return jnp.where(rowq == colq, tiled, 0.0)


def _tile4(row):
  return jnp.concatenate([row] * 4, axis=1)


def _dinv_packed(deg_ref, n_real, n4):
  deg = 1.0 + deg_ref[0] + deg_ref[1]
  riota = lax.broadcasted_iota(jnp.int32, (n4, 1), 0)
  cols = []
  for q in range(4):
    dq = deg[:, DEGW * q:DEGW * q + 1]
    valid = riota * 4 + q < n_real
    dq = jnp.where(valid, lax.rsqrt(dq), 0.0)
    cols.append(jnp.broadcast_to(dq, (n4, 32)))
  return jnp.concatenate(cols, axis=1)


def _tc_prologue(n_real, npad):
  n4 = npad // 4

  def body(x4_ref, w1_ref, b1_ref, deg_ref,
           hplo_ref, hphi_ref, stlo_ref, sthi_ref, dinv_ref):
    x4 = x4_ref[...]
    w1 = w1_ref[...]
    hh = w1.shape[1] // 2
    dinv_p = _dinv_packed(deg_ref, n_real, n4)
    deginv_p = dinv_p * dinv_p
    h1lo = jnp.dot(x4, _blockdiag4(w1[:, :hh]),
                   preferred_element_type=jnp.float32, precision=PREC)
    h1hi = jnp.dot(x4, _blockdiag4(w1[:, hh:]),
                   preferred_element_type=jnp.float32, precision=PREC)
    b1 = b1_ref[...]
    hplo_ref[...] = dinv_p * h1lo
    hphi_ref[...] = dinv_p * h1hi
    stlo_ref[...] = deginv_p * h1lo + _tile4(b1[:, :hh])
    sthi_ref[...] = deginv_p * h1hi + _tile4(b1[:, hh:])
    dinv_ref[...] = dinv_p

  def call(x4, w1, b1_2d, deg4):
    return pl.pallas_call(
        body,
        out_shape=[jax.ShapeDtypeStruct((n4, 128), jnp.float32)
                   for _ in range(5)],
        name="tc_prologue",
        compiler_params=pltpu.CompilerParams(vmem_limit_bytes=100 * 1024 * 1024),
    )(x4, w1, b1_2d, deg4)

  return call


def _tc_mid(npad):
  n4 = npad // 4

  def body(acclo_ref, acchi_ref, stlo_ref, sthi_ref, dinv_ref, w2_ref,
           b2_ref, hplo_ref, hphi_ref, st2lo_ref, st2hi_ref):
    acclo = acclo_ref[...]
    acchi = acchi_ref[...]
    dinv_p = dinv_ref[...]
    deginv_p = dinv_p * dinv_p
    w2 = w2_ref[...]
    hh = w2.shape[1] // 2
    out1lo = jnp.maximum(dinv_p * (acclo[0] + acclo[1]) + stlo_ref[...], 0.0)
    out1hi = jnp.maximum(dinv_p * (acchi[0] + acchi[1]) + sthi_ref[...], 0.0)
    h2lo = (jnp.dot(out1lo, _blockdiag4(w2[:hh, :hh]),
                    preferred_element_type=jnp.float32, precision=PREC)
            + jnp.dot(out1hi, _blockdiag4(w2[hh:, :hh]),
                      preferred_element_type=jnp.float32, precision=PREC))
    h2hi = (jnp.dot(out1lo, _blockdiag4(w2[:hh, hh:]),
                    preferred_element_type=jnp.float32, precision=PREC)
            + jnp.dot(out1hi, _blockdiag4(w2[hh:, hh:]),
                      preferred_element_type=jnp.float32, precision=PREC))
    b2 = b2_ref[...]
    hplo_ref[...] = dinv_p * h2lo
    hphi_ref[...] = dinv_p * h2hi
    st2lo_ref[...] = deginv_p * h2lo + _tile4(b2[:, :hh])
    st2hi_ref[...] = deginv_p * h2hi + _tile4(b2[:, hh:])

  def call(acc_lo4, acc_hi4, stlo, sthi, dinvp, w2, b2_2d):
    return pl.pallas_call(
        body,
        out_shape=[jax.ShapeDtypeStruct((n4, 128), jnp.float32)
                   for _ in range(4)],
        name="tc_mid",
        compiler_params=pltpu.CompilerParams(vmem_limit_bytes=100 * 1024 * 1024),
    )(acc_lo4, acc_hi4, stlo, sthi, dinvp, w2, b2_2d)

  return call


def _tc_head(npad, g):
  n4 = npad // 4

  def body(acclo_ref, acchi_ref, st2lo_ref, st2hi_ref, dinv_ref, batchq_ref,
           wfc_ref, bfc_ref, y_ref):
    acclo = acclo_ref[...]
    acchi = acchi_ref[...]
    dinv_p = dinv_ref[...]
    out2lo = jnp.maximum(dinv_p * (acclo[0] + acclo[1]) + st2lo_ref[...], 0.0)
    out2hi = jnp.maximum(dinv_p * (acchi[0] + acchi[1]) + st2hi_ref[...], 0.0)
    gids = lax.broadcasted_iota(jnp.int32, (g, n4), 0)
    pooled_lo = jnp.zeros((g, 32), jnp.float32)
    pooled_hi = jnp.zeros((g, 32), jnp.float32)
    cnt = jnp.zeros((g, 1), jnp.float32)
    for q in range(4):
      oh = jnp.where(gids == batchq_ref[q:q + 1, :], 1.0, 0.0)
      mlo = jnp.dot(oh, out2lo, preferred_element_type=jnp.float32,
                    precision=PREC)
      mhi = jnp.dot(oh, out2hi, preferred_element_type=jnp.float32,
                    precision=PREC)
      pooled_lo = pooled_lo + mlo[:, 32 * q:32 * q + 32]
      pooled_hi = pooled_hi + mhi[:, 32 * q:32 * q + 32]
      cnt = cnt + jnp.sum(oh, axis=1, keepdims=True)
    pooled = jnp.concatenate([pooled_lo, pooled_hi], axis=1)
    pooled = pooled / jnp.maximum(cnt, 1.0)
    y_ref[...] = (
        jnp.dot(pooled, wfc_ref[...], preferred_element_type=jnp.float32,
                precision=PREC)
        + bfc_ref[...])

  def call(acc_lo4, acc_hi4, st2lo, st2hi, dinvp, batchq, wfc, bfc_2d):
    out = wfc.shape[1]
    return pl.pallas_call(
        body,
        out_shape=jax.ShapeDtypeStruct((g, out), jnp.float32),
        name="tc_head",
        compiler_params=pltpu.CompilerParams(vmem_limit_bytes=100 * 1024 * 1024),
    )(acc_lo4, acc_hi4, st2lo, st2hi, dinvp, batchq, wfc, bfc_2d)

  return call


def kernel(x, edge_index, batch, W1, b1, W2, b2, Wfc, bfc):
  n, din = x.shape
  e = edge_index.shape[1]
  h = W1.shape[1]
  g = 128

  npad = int(np.ceil(n / (NS * C))) * NS * C
  k = NBUF * int(np.ceil(e / (NW * C * NBUF)))
  cap = NW * k * C
  padrow = npad - 1

  ei_r = jnp.pad(edge_index, ((0, 0), (0, cap - e)),
                 constant_values=padrow).reshape(2, NW, k, C)

  n4 = npad // 4
  hh = h // 2
  x4 = jnp.zeros((npad, din), x.dtype).at[:n].set(x).reshape(n4, 4 * din)
  batch_pad = jnp.full((npad,), g, jnp.int32).at[:n].set(batch)
  batchq = batch_pad.reshape(n4, 4).T
  b1_2d = b1.reshape(1, -1)
  b2_2d = b2.reshape(1, -1)
  bfc_2d = bfc.reshape(1, -1)

  deg_fn = _make_deg_kernel(npad, k)
  mp_fn = _make_mp_kernel(npad, k, hh)

  def unpack(t):
    return t.reshape(npad, hh)

  def pack(a):
    return a.reshape(NC, n4, 128)

  degtab = deg_fn(ei_r)
  deg4 = degtab.reshape(NC, n4, 4 * DEGW)
  hp1_lo, hp1_hi, st1_lo, st1_hi, dinvp = _tc_prologue(n, npad)(
      x4, W1, b1_2d, deg4)
  acc1_lo = pack(mp_fn(unpack(hp1_lo), ei_r))
  acc1_hi = pack(mp_fn(unpack(hp1_hi), ei_r))
  hp2_lo, hp2_hi, st2_lo, st2_hi = _tc_mid(npad)(
      acc1_lo, acc1_hi, st1_lo, st1_hi, dinvp, W2, b2_2d)
  acc2_lo = pack(mp_fn(unpack(hp2_lo), ei_r))
  acc2_hi = pack(mp_fn(unpack(hp2_hi), ei_r))
  return _tc_head(npad, g)(acc2_lo, acc2_hi, st2_lo, st2_hi, dinvp, batchq,
                           Wfc, bfc_2d)
